# Initial kernel scaffold; baseline (speedup 1.0000x reference)
import jax
import jax.numpy as jnp
from jax.experimental import pallas as pl


def kernel(x, edge_index, batch, W1, a_src1, a_dst1, b1, W2, a_src2, a_dst2, b2):
    raise NotImplementedError("write your pallas kernel here")


# speedup vs baseline: 2.2920x; 2.2920x over previous
import functools

import jax
import jax.numpy as jnp
from jax import lax
from jax.experimental import pallas as pl
from jax.experimental.pallas import tpu as pltpu
from jax.experimental.pallas import tpu_sc as plsc

N = 10000
E = 160000
D = 128
HEADS = 8
C1 = 64
F1 = HEADS * C1
C2 = 16
G = 64

NC, NS = 2, 16
N_PAD = 10240
E_PAD = 163840
B = 128
ROW1 = 80
ROW2 = 32
NEG = -1e30
BN = 2048
BN2 = 1024
HIGH = lax.Precision.HIGHEST

f32 = jnp.float32
i32 = jnp.int32


def _lrelu(x):
    return jnp.where(x >= 0, x, 0.2 * x)


def _tca_body(x_ref, w1_ref, as1_ref, ad1_ref,
              hh_ref, ast_ref, adt_ref, asn_ref, adn_ref, maxs_ref, mscr):
    i = pl.program_id(0)
    row0 = i * BN
    mcol = row0 + lax.broadcasted_iota(i32, (BN, 1), 0) < N
    mrow = row0 + lax.broadcasted_iota(i32, (1, BN), 1) < N
    hb = jnp.dot(x_ref[...], w1_ref[...], preferred_element_type=f32,
                 precision=HIGH)
    for h in range(HEADS):
        hh_ref[h] = hb[:, h * C1:(h + 1) * C1]
    asb = jnp.dot(hb, as1_ref[...], preferred_element_type=f32, precision=HIGH)
    adb = jnp.dot(hb, ad1_ref[...], preferred_element_type=f32, precision=HIGH)
    asb = jnp.where(mcol, asb, NEG)
    adb = jnp.where(mcol, adb, NEG)
    asn_ref[...] = asb
    adn_ref[...] = adb
    dn = (((0,), (1,)), ((), ()))
    astb = lax.dot_general(as1_ref[...], hb, dn,
                           preferred_element_type=f32, precision=HIGH)
    adtb = lax.dot_general(ad1_ref[...], hb, dn,
                           preferred_element_type=f32, precision=HIGH)
    ast_ref[...] = jnp.where(mrow, astb, NEG)
    adt_ref[...] = jnp.where(mrow, adtb, NEG)

    @pl.when(i == 0)
    def _():
        mscr[...] = jnp.full((1, HEADS), NEG, f32)
    mscr[...] = jnp.maximum(mscr[...], jnp.max(asb, axis=0, keepdims=True))
    maxs_ref[...] = mscr[...]


def _tc_a(x, W1, As1, Ad1):
    nb = N_PAD // BN
    return pl.pallas_call(
        _tca_body,
        grid=(nb,),
        in_specs=[
            pl.BlockSpec((BN, D), lambda i: (i, 0)),
            pl.BlockSpec((D, F1), lambda i: (0, 0)),
            pl.BlockSpec((F1, HEADS), lambda i: (0, 0)),
            pl.BlockSpec((F1, HEADS), lambda i: (0, 0)),
        ],
        out_specs=[
            pl.BlockSpec((HEADS, BN, C1), lambda i: (0, i, 0)),
            pl.BlockSpec((HEADS, BN), lambda i: (0, i)),
            pl.BlockSpec((HEADS, BN), lambda i: (0, i)),
            pl.BlockSpec((BN, HEADS), lambda i: (i, 0)),
            pl.BlockSpec((BN, HEADS), lambda i: (i, 0)),
            pl.BlockSpec((1, HEADS), lambda i: (0, 0)),
        ],
        out_shape=[
            jax.ShapeDtypeStruct((HEADS, N_PAD, C1), f32),
            jax.ShapeDtypeStruct((HEADS, N_PAD), f32),
            jax.ShapeDtypeStruct((HEADS, N_PAD), f32),
            jax.ShapeDtypeStruct((N_PAD, HEADS), f32),
            jax.ShapeDtypeStruct((N_PAD, HEADS), f32),
            jax.ShapeDtypeStruct((1, HEADS), f32),
        ],
        scratch_shapes=[pltpu.VMEM((1, HEADS), f32)],
    )(x, W1, As1, Ad1)


def _tcb_body(acc1_ref, hh_ref, asn_ref, adn_ref, maxs_ref, w2_ref, b1_ref,
              as2w_ref, ad2w_ref,
              h2_ref, as2_ref, ad2_ref, maxs2_ref, m2scr):
    i = pl.program_id(0)
    asb = asn_ref[...]
    adb = adn_ref[...]
    ws = jnp.exp(_lrelu(asb + adb) - _lrelu(maxs_ref[...] + adb))
    a2 = acc1_ref[...]
    STR = (NCHUNK + 1) * 16
    parts = []
    for h in range(HEADS):
        wcol = ws[:, h:h + 1]
        num = a2[:, h * STR:h * STR + C1] + wcol * hh_ref[h]
        den = a2[:, h * STR + C1:h * STR + C1 + 1] + wcol + 1e-16
        parts.append(num / den)
    o1 = jnp.concatenate(parts, axis=1) + b1_ref[...]
    h1 = jnp.where(o1 > 0, o1, jnp.exp(jnp.minimum(o1, 0.0)) - 1.0)
    h2b = jnp.dot(h1, w2_ref[...], preferred_element_type=f32, precision=HIGH)
    h2_ref[...] = h2b
    as2b = jnp.dot(h2b, as2w_ref[...], preferred_element_type=f32,
                   precision=HIGH)
    ad2b = jnp.dot(h2b, ad2w_ref[...], preferred_element_type=f32,
                   precision=HIGH)
    mcol = i * BN2 + lax.broadcasted_iota(i32, (BN2, 1), 0) < N
    as2b = jnp.where(mcol, as2b, NEG)
    ad2b = jnp.where(mcol, ad2b, NEG)
    as2_ref[...] = as2b
    ad2_ref[...] = ad2b

    @pl.when(i == 0)
    def _():
        m2scr[...] = jnp.full((1, HEADS), NEG, f32)
    m2scr[...] = jnp.maximum(m2scr[...], jnp.max(as2b))
    maxs2_ref[...] = m2scr[...]


def _tc_b(acc1, hh, asn, adn, maxs, W2, b1r, as2w, ad2w):
    nb = N_PAD // BN2
    return pl.pallas_call(
        _tcb_body,
        grid=(nb,),
        in_specs=[
            pl.BlockSpec((BN2, HEADS * (NCHUNK + 1) * 16), lambda i: (i, 0)),
            pl.BlockSpec((HEADS, BN2, C1), lambda i: (0, i, 0)),
            pl.BlockSpec((BN2, HEADS), lambda i: (i, 0)),
            pl.BlockSpec((BN2, HEADS), lambda i: (i, 0)),
            pl.BlockSpec((1, HEADS), lambda i: (0, 0)),
            pl.BlockSpec((F1, C2), lambda i: (0, 0)),
            pl.BlockSpec((1, F1), lambda i: (0, 0)),
            pl.BlockSpec((C2, 1), lambda i: (0, 0)),
            pl.BlockSpec((C2, 1), lambda i: (0, 0)),
        ],
        out_specs=[
            pl.BlockSpec((BN2, C2), lambda i: (i, 0)),
            pl.BlockSpec((BN2, 1), lambda i: (i, 0)),
            pl.BlockSpec((BN2, 1), lambda i: (i, 0)),
            pl.BlockSpec((1, HEADS), lambda i: (0, 0)),
        ],
        out_shape=[
            jax.ShapeDtypeStruct((N_PAD, C2), f32),
            jax.ShapeDtypeStruct((N_PAD, 1), f32),
            jax.ShapeDtypeStruct((N_PAD, 1), f32),
            jax.ShapeDtypeStruct((1, HEADS), f32),
        ],
        scratch_shapes=[pltpu.VMEM((1, HEADS), f32)],
    )(acc1, hh, asn, adn, maxs, W2, b1r, as2w, ad2w)


def _tcc_body(acc2_ref, h2_ref, as2_ref, ad2_ref, maxs2_ref, b2_ref,
              batch_ref, out_ref, sums, csum):
    i = pl.program_id(0)
    as2b = as2_ref[...]
    ad2b = ad2_ref[...]
    m2 = maxs2_ref[...][:, 0:1]
    w2s = jnp.exp(_lrelu(as2b + ad2b) - _lrelu(m2 + ad2b))
    num = acc2_ref[0][:, 0:C2] + acc2_ref[1][:, 0:C2] + w2s * h2_ref[...]
    den = (acc2_ref[0][:, C2:C2 + 1] + acc2_ref[1][:, C2:C2 + 1]
           + w2s + 1e-16)
    out2 = num / den + b2_ref[...]
    bb = batch_ref[...]
    ids = lax.broadcasted_iota(i32, (G, BN), 0).astype(f32)
    onehot = (ids == bb).astype(f32)

    @pl.when(i == 0)
    def _():
        sums[...] = jnp.zeros((G, C2), f32)
        csum[...] = jnp.zeros((G, 128), f32)
    sums[...] = sums[...] + jnp.dot(onehot, out2, preferred_element_type=f32,
                                    precision=HIGH)
    csum[...] = csum[...] + jnp.sum(onehot, axis=1, keepdims=True)
    out_ref[...] = sums[...] / jnp.maximum(csum[...][:, 0:C2], 1.0)


def _tc_c(acc2, h2, as2, ad2, maxs2, b2r, batchf):
    nb = N_PAD // BN
    return pl.pallas_call(
        _tcc_body,
        grid=(nb,),
        in_specs=[
            pl.BlockSpec((NC, BN, ROW2), lambda i: (0, i, 0)),
            pl.BlockSpec((BN, C2), lambda i: (i, 0)),
            pl.BlockSpec((BN, 1), lambda i: (i, 0)),
            pl.BlockSpec((BN, 1), lambda i: (i, 0)),
            pl.BlockSpec((1, HEADS), lambda i: (0, 0)),
            pl.BlockSpec((1, C2), lambda i: (0, 0)),
            pl.BlockSpec((1, BN), lambda i: (0, i)),
        ],
        out_specs=pl.BlockSpec((G, C2), lambda i: (0, 0)),
        out_shape=jax.ShapeDtypeStruct((G, C2), f32),
        scratch_shapes=[pltpu.VMEM((G, C2), f32), pltpu.VMEM((G, 128), f32)],
    )(acc2, h2, as2, ad2, maxs2, b2r, batchf)


NCHUNK = C1 // 16


def _sc1_body(*refs):
    tabs = refs[0:16]
    dstp, wv1, gidxc = refs[16:19]
    acc_out = refs[19]
    (didx, gidx, rows16, wbuf, wload, wfull,
     o0, o1, o2, o3, wl0, zbuf,
     a0, a1, a2, a3, aw, sem) = refs[20:]
    accs = [a0, a1, a2, a3]
    obufs = [o0, o1, o2, o3]
    c = lax.axis_index("c")
    s = lax.axis_index("s")
    lane0 = jnp.where(lax.iota(i32, 16) == 0, 1.0, 0.0).astype(f32)
    zrow = jnp.zeros((16,), f32)
    nrows_t = N_PAD // NS
    npass = HEADS // NC
    epb = E_PAD // NS
    nbatch = epb // B
    STR = (NCHUNK + 1) * 16

    def zb(i, _):
        zbuf[i, pl.ds(0, 16)] = zrow
        return 0
    lax.fori_loop(0, B, zb, 0)

    for p in range(npass):
        hd = c * npass + p
        for acc in accs + [aw]:
            for z in range(nrows_t // B):
                pltpu.sync_copy(zbuf, acc.at[pl.ds(s * nrows_t + z * B, B)])
        plsc.subcore_barrier()

        def batch_body(b, _):
            base = s * epb + b * B
            pltpu.sync_copy(dstp.at[pl.ds(base, B)], didx)
            pltpu.sync_copy(wv1.at[pl.ds(hd * E_PAD + base, B)], wload)
            pltpu.sync_copy(gidxc.at[pl.ds(c * E_PAD + base, B)], gidx)
            for g in range(B // 16):
                wbuf[...] = wload[pl.ds(g * 16, 16)]
                for j in range(16):
                    e = g * 16 + j
                    wsp = plsc.load_gather(wbuf, [jnp.full((16,), j, i32)])
                    wfull[e] = wsp
                    wl0[e] = wsp * lane0
            for k in range(NCHUNK):
                pltpu.async_copy(tabs[p * NCHUNK + k].at[gidx], rows16,
                                 sem).wait()
                for e in range(B):
                    obufs[k][e, pl.ds(0, 16)] = (rows16[e, pl.ds(0, 16)]
                                                 * wfull[e])
                pltpu.sync_copy(obufs[k], accs[k].at[didx], add=True)
            pltpu.sync_copy(wl0, aw.at[didx], add=True)
            plsc.subcore_barrier()
            return 0
        lax.fori_loop(0, nbatch, batch_body, 0)
        plsc.subcore_barrier()
        for k in range(NCHUNK):
            pltpu.async_copy(accs[k].at[pl.ds(s * nrows_t, nrows_t)],
                             acc_out.at[pl.ds(s * nrows_t, nrows_t),
                                        pl.ds(hd * STR + k * 16, 16)],
                             sem).wait()
        pltpu.async_copy(aw.at[pl.ds(s * nrows_t, nrows_t)],
                         acc_out.at[pl.ds(s * nrows_t, nrows_t),
                                    pl.ds(hd * STR + NCHUNK * 16, 16)],
                         sem).wait()
        plsc.subcore_barrier()


def _sc_l1(tabs, dstp, wv1, gidxc):
    mesh = plsc.VectorSubcoreMesh(core_axis_name="c", subcore_axis_name="s",
                                  num_cores=NC, num_subcores=NS)
    return pl.kernel(
        _sc1_body,
        out_type=jax.ShapeDtypeStruct((N_PAD, HEADS * (NCHUNK + 1) * 16),
                                      f32),
        mesh=mesh,
        compiler_params=pltpu.CompilerParams(needs_layout_passes=False, use_tc_tiling_on_sc=False),
        scratch_types=[
            pltpu.VMEM((B,), i32),
            pltpu.VMEM((B,), i32),
            pltpu.VMEM((B, 16), f32),
            pltpu.VMEM((16,), f32),
            pltpu.VMEM((B,), f32),
            pltpu.VMEM((B, 16), f32),
            pltpu.VMEM((B, 16), f32),
            pltpu.VMEM((B, 16), f32),
            pltpu.VMEM((B, 16), f32),
            pltpu.VMEM((B, 16), f32),
            pltpu.VMEM((B, 16), f32),
            pltpu.VMEM((B, 16), f32),
            pltpu.VMEM_SHARED((N_PAD, 16), f32),
            pltpu.VMEM_SHARED((N_PAD, 16), f32),
            pltpu.VMEM_SHARED((N_PAD, 16), f32),
            pltpu.VMEM_SHARED((N_PAD, 16), f32),
            pltpu.VMEM_SHARED((N_PAD, 16), f32),
            pltpu.SemaphoreType.DMA,
        ],
    )(*tabs, dstp, wv1, gidxc)


def _sc2_body(h2, as2f, ad2f, srcp, dstp, wv, acc_out,
              as_v, ad_v, sidx, didx, rows, orows, wbuf, wload,
              zbuf, acc_sh, sem):
    c = lax.axis_index("c")
    s = lax.axis_index("s")
    zrow = jnp.zeros((16,), f32)
    lane0 = jnp.where(lax.iota(i32, 16) == 0, 1.0, 0.0).astype(f32)
    nrows_t = N_PAD // NS

    def zb(i, _):
        for k in range(ROW2 // 16):
            zbuf[i, pl.ds(k * 16, 16)] = zrow
        return 0
    lax.fori_loop(0, B, zb, 0)

    for z in range(nrows_t // B):
        pltpu.sync_copy(zbuf, acc_sh.at[pl.ds(s * nrows_t + z * B, B)])
    pltpu.sync_copy(as2f.at[pl.ds(0, N_PAD)], as_v)
    pltpu.sync_copy(ad2f.at[pl.ds(0, N_PAD)], ad_v)
    plsc.subcore_barrier()

    def mx(i, m):
        return jnp.maximum(m, as_v[pl.ds(i * 16, 16)])
    m16 = lax.fori_loop(0, N_PAD // 16, mx, jnp.full((16,), NEG, f32))
    lanes = lax.iota(i32, 16)
    msv = m16
    for sh in (1, 2, 4, 8):
        wbuf[...] = msv
        msv = jnp.maximum(msv, plsc.load_gather(wbuf, [lanes ^ sh]))

    t_id = c * NS + s
    epb = E_PAD // (NC * NS)
    nbatch = epb // B
    base0 = t_id * epb

    def batch_body(b, _):
        base = base0 + b * B
        pltpu.sync_copy(srcp.at[pl.ds(base, B)], sidx)
        pltpu.sync_copy(dstp.at[pl.ds(base, B)], didx)
        pltpu.sync_copy(wv.at[pl.ds(base, B)], wload)
        pltpu.async_copy(h2.at[sidx], rows, sem).wait()
        for g in range(B // 16):
            sl = pl.ds(g * 16, 16)
            wbuf[...] = wload[sl]
            for j in range(16):
                e = g * 16 + j
                wsp = plsc.load_gather(wbuf, [jnp.full((16,), j, i32)])
                orows[e, pl.ds(0, 16)] = rows[e, :] * wsp
                orows[e, pl.ds(C2, 16)] = wsp * lane0
        pltpu.sync_copy(orows, acc_sh.at[didx], add=True)
        return 0
    lax.fori_loop(0, nbatch, batch_body, 0)
    plsc.subcore_barrier()
    pltpu.sync_copy(acc_sh.at[pl.ds(s * nrows_t, nrows_t)],
                    acc_out.at[pl.ds(c * N_PAD + s * nrows_t, nrows_t)])


def _sc_l2(h2, as2f, ad2f, srcp, dstp, wv):
    mesh = plsc.VectorSubcoreMesh(core_axis_name="c", subcore_axis_name="s",
                                  num_cores=NC, num_subcores=NS)
    return pl.kernel(
        _sc2_body,
        out_type=jax.ShapeDtypeStruct((NC * N_PAD, ROW2), f32),
        mesh=mesh,
        compiler_params=pltpu.CompilerParams(needs_layout_passes=False, use_tc_tiling_on_sc=False),
        scratch_types=[
            pltpu.VMEM((N_PAD,), f32),
            pltpu.VMEM((N_PAD,), f32),
            pltpu.VMEM((B,), i32),
            pltpu.VMEM((B,), i32),
            pltpu.VMEM((B, C2), f32),
            pltpu.VMEM((B, ROW2), f32),
            pltpu.VMEM((16,), f32),
            pltpu.VMEM((B,), f32),
            pltpu.VMEM((B, ROW2), f32),
            pltpu.VMEM_SHARED((N_PAD, ROW2), f32),
            pltpu.SemaphoreType.DMA,
        ],
    )(h2, as2f, ad2f, srcp, dstp, wv)


def kernel(x, edge_index, batch, W1, a_src1, a_dst1, b1, W2, a_src2, a_dst2,
           b2):
    src = edge_index[0].astype(i32)
    dst = edge_index[1].astype(i32)
    srcp = jnp.concatenate([src, jnp.zeros((E_PAD - E,), i32)])
    dstp = jnp.concatenate([dst, jnp.full((E_PAD - E,), N_PAD - 1, i32)])

    eye = jnp.eye(HEADS, dtype=f32)
    As1 = (a_src1[:, :, None] * eye[:, None, :]).reshape(F1, HEADS)
    Ad1 = (a_dst1[:, :, None] * eye[:, None, :]).reshape(F1, HEADS)

    xp = jnp.pad(x, ((0, N_PAD - N), (0, 0)))
    hh, asT, adT, asn, adn, maxs = _tc_a(xp, W1, As1, Ad1)

    ms1 = jnp.max(asT, axis=1)
    wv1h = jnp.exp(_lrelu(asT[:, srcp] + adT[:, dstp])
                   - _lrelu(ms1[:, None] + adT[:, dstp]))
    cols = []
    for h in range(HEADS):
        msg = wv1h[h][:, None] * hh[h][srcp]
        seg = jax.ops.segment_sum(msg, dstp, num_segments=N_PAD)
        wsum = jax.ops.segment_sum(wv1h[h], dstp, num_segments=N_PAD)
        cols.append(jnp.concatenate([seg, wsum[:, None],
                                     jnp.zeros((N_PAD, 15), f32)], axis=1))
    acc1 = jnp.concatenate(cols, axis=1)

    b1r = b1.reshape(1, F1)
    as2w = a_src2.reshape(C2, 1)
    ad2w = a_dst2.reshape(C2, 1)
    h2, as2, ad2, maxs2 = _tc_b(acc1, hh, asn, adn, maxs, W2, b1r, as2w, ad2w)

    as2f = as2.reshape(-1)
    ad2f = ad2.reshape(-1)
    ms2 = jnp.max(as2f)
    wv = jnp.exp(_lrelu(as2f[srcp] + ad2f[dstp])
                 - _lrelu(ms2 + ad2f[dstp]))
    acc2 = _sc_l2(h2, as2f, ad2f, srcp, dstp,
                  wv).reshape(NC, N_PAD, ROW2)

    b2r = b2.reshape(1, C2)
    batchf = jnp.concatenate(
        [batch.astype(f32), jnp.full((N_PAD - N,), 100.0, f32)]
    ).reshape(1, N_PAD)
    return _tc_c(acc2, h2, as2, ad2, maxs2, b2r, batchf)
